# baseline (speedup 1.0000x reference)
import jax
import jax.numpy as jnp
from jax import lax
from jax.experimental import pallas as pl
from jax.experimental.pallas import tpu as pltpu
from jax.experimental.pallas import tpu_sc as plsc

B = 16384
D = 64
NC = 2
NS = 16
NW = NC * NS
BPW = B // NW


def _rsqrt(x):
    i = lax.bitcast_convert_type(x, jnp.int32)
    i = jnp.int32(0x5F3759DF) - (i >> 1)
    y = lax.bitcast_convert_type(i, jnp.float32)
    for _ in range(3):
        y = y * (1.5 - 0.5 * x * y * y)
    return y


def _tec_body(hid_hbm, lab_hbm, tid_hbm, entT_hbm, relT_hbm, out_hbm,
              hidx, lidx, tidx, hcol, tcol, rcol, outv, sem):
    wid = lax.axis_index("s") * NC + lax.axis_index("c")
    base = wid * BPW

    pltpu.sync_copy(hid_hbm.at[pl.ds(base, BPW)], hidx)
    pltpu.sync_copy(lab_hbm.at[pl.ds(base, BPW)], lidx)
    pltpu.sync_copy(tid_hbm.at[pl.ds(base, BPW)], tidx)

    cps = []
    for d in range(D):
        dst = pl.ds(d * BPW, BPW)
        cps.append(pltpu.async_copy(entT_hbm.at[d].at[hidx], hcol.at[dst], sem))
        cps.append(pltpu.async_copy(entT_hbm.at[d].at[tidx], tcol.at[dst], sem))
        cps.append(pltpu.async_copy(relT_hbm.at[d].at[lidx], rcol.at[dst], sem))
    for cp in cps:
        cp.wait()

    zero16 = jnp.zeros((16,), jnp.float32)

    def block(blk, carry):
        r0 = blk * 16
        hh = tt = rr = hr = ht = tr = zero16
        for d in range(D):
            hd = hcol[pl.ds(d * BPW + r0, 16)]
            td = tcol[pl.ds(d * BPW + r0, 16)]
            rd = rcol[pl.ds(d * BPW + r0, 16)]
            hh = hh + hd * hd
            tt = tt + td * td
            rr = rr + rd * rd
            hr = hr + hd * rd
            ht = ht + hd * td
            tr = tr + td * rd
        ih = _rsqrt(jnp.maximum(hh, 1e-24))
        it = _rsqrt(jnp.maximum(tt, 1e-24))
        ssd = (hh * ih * ih + rr + tt * it * it
               + 2.0 * (hr * ih) - 2.0 * (ht * (ih * it)) - 2.0 * (tr * it))
        ssd = jnp.maximum(ssd, 0.0)
        outv[pl.ds(r0, 16)] = ssd * _rsqrt(jnp.maximum(ssd, 1e-24))
        return carry

    lax.fori_loop(0, BPW // 16, block, 0)

    pltpu.sync_copy(outv, out_hbm.at[pl.ds(base, BPW)])


@jax.jit
def _sc_transe(hid, lab, tid, ent_t, rel_t):
    mesh = plsc.VectorSubcoreMesh(core_axis_name="c", subcore_axis_name="s")
    f = pl.kernel(
        _tec_body,
        mesh=mesh,
        out_type=jax.ShapeDtypeStruct((B,), jnp.float32),
        compiler_params=pltpu.CompilerParams(use_tc_tiling_on_sc=False),
        scratch_types=[
            pltpu.VMEM((BPW,), jnp.int32),
            pltpu.VMEM((BPW,), jnp.int32),
            pltpu.VMEM((BPW,), jnp.int32),
            pltpu.VMEM((D * BPW,), jnp.float32),
            pltpu.VMEM((D * BPW,), jnp.float32),
            pltpu.VMEM((D * BPW,), jnp.float32),
            pltpu.VMEM((BPW,), jnp.float32),
            pltpu.SemaphoreType.DMA,
        ],
    )
    return f(hid, lab, tid, ent_t, rel_t)


def kernel(head_ind, label, tail_ind, ent_embs, rel_embs):
    hid = head_ind.astype(jnp.int32)
    lab = label.astype(jnp.int32)
    tid = tail_ind.astype(jnp.int32)
    return _sc_transe(hid, lab, tid, ent_embs.T, rel_embs.T)


# speedup vs baseline: 7.8234x; 7.8234x over previous
import jax
import jax.numpy as jnp
from jax import lax
from jax.experimental import pallas as pl
from jax.experimental.pallas import tpu as pltpu
from jax.experimental.pallas import tpu_sc as plsc

B = 16384
D = 64
NC = 2
NS = 16
NW = NC * NS
BPW = B // NW
CH = 128
NCH = BPW // CH


def _rsqrt(x):
    i = lax.bitcast_convert_type(x, jnp.int32)
    i = jnp.int32(0x5F3759DF) - (i >> 1)
    y = lax.bitcast_convert_type(i, jnp.float32)
    for _ in range(3):
        y = y * (1.5 - 0.5 * x * y * y)
    return y


def _tec_body(hid_hbm, lab_hbm, tid_hbm, ent2_hbm, rel2_hbm, out_hbm,
              hidx, lidx, tidx, hrows, trows, rrows, outv, sem):
    wid = lax.axis_index("s") * NC + lax.axis_index("c")
    base = wid * BPW

    pltpu.sync_copy(hid_hbm.at[pl.ds(base, BPW)], hidx)
    pltpu.sync_copy(lab_hbm.at[pl.ds(base, BPW)], lidx)
    pltpu.sync_copy(tid_hbm.at[pl.ds(base, BPW)], tidx)

    lane = lax.broadcasted_iota(jnp.int32, (16,), 0)
    zero16 = jnp.zeros((16,), jnp.float32)
    bfly = [lane ^ k for k in (8, 4, 2, 1)]

    def hsum(x):
        for idx in bfly:
            x = x + x.at[idx].get(mode="promise_in_bounds", unique_indices=True)
        return x

    def compute(ch, hbit, lbit, tbit):
        for b in range(CH // 16):
            acc = [zero16] * 6
            for j in range(16):
                i = b * 16 + j
                hlo = [hrows[i, pl.ds(16 * c, 16)] for c in range(4)]
                hhi = [hrows[i, pl.ds(64 + 16 * c, 16)] for c in range(4)]
                tlo = [trows[i, pl.ds(16 * c, 16)] for c in range(4)]
                thi = [trows[i, pl.ds(64 + 16 * c, 16)] for c in range(4)]
                rlo = [rrows[i, pl.ds(16 * c, 16)] for c in range(4)]
                rhi = [rrows[i, pl.ds(64 + 16 * c, 16)] for c in range(4)]
                hb = hbit[b][j] != 0
                tb = tbit[b][j] != 0
                rb = lbit[b][j] != 0
                h = [jnp.where(hb, a, o) for a, o in zip(hhi, hlo)]
                t = [jnp.where(tb, a, o) for a, o in zip(thi, tlo)]
                r = [jnp.where(rb, a, o) for a, o in zip(rhi, rlo)]
                prods = [
                    sum(h[c] * h[c] for c in range(4)),
                    sum(t[c] * t[c] for c in range(4)),
                    sum(r[c] * r[c] for c in range(4)),
                    sum(h[c] * r[c] for c in range(4)),
                    sum(h[c] * t[c] for c in range(4)),
                    sum(t[c] * r[c] for c in range(4)),
                ]
                m = lane == j
                acc = [jnp.where(m, hsum(p), a) for p, a in zip(prods, acc)]
            hh, tt, rr, hr, ht, tr = acc
            ih = _rsqrt(jnp.maximum(hh, 1e-24))
            it = _rsqrt(jnp.maximum(tt, 1e-24))
            ssd = (hh * ih * ih + rr + tt * it * it
                   + 2.0 * (hr * ih) - 2.0 * (ht * (ih * it)) - 2.0 * (tr * it))
            ssd = jnp.maximum(ssd, 0.0)
            outv[pl.ds(ch * CH + b * 16, 16)] = ssd * _rsqrt(jnp.maximum(ssd, 1e-24))

    def chunk(ch, carry):
        hbit, lbit, tbit = [], [], []
        for b in range(CH // 16):
            s = pl.ds(ch * CH + b * 16, 16)
            hv, lv, tv = hidx[s], lidx[s], tidx[s]
            hbit.append(hv & 1)
            lbit.append(lv & 1)
            tbit.append(tv & 1)
            hidx[s] = hv >> 1
            lidx[s] = lv >> 1
            tidx[s] = tv >> 1
        sl = pl.ds(ch * CH, CH)
        cph = pltpu.async_copy(ent2_hbm.at[hidx.at[sl]], hrows, sem)
        cpt = pltpu.async_copy(ent2_hbm.at[tidx.at[sl]], trows, sem)
        cpr = pltpu.async_copy(rel2_hbm.at[lidx.at[sl]], rrows, sem)
        cph.wait()
        cpt.wait()
        cpr.wait()
        compute(ch, hbit, lbit, tbit)
        return carry

    lax.fori_loop(0, NCH, chunk, 0)

    pltpu.sync_copy(outv, out_hbm.at[pl.ds(base, BPW)])


@jax.jit
def _sc_transe(hid, lab, tid, ent2, rel2):
    mesh = plsc.VectorSubcoreMesh(core_axis_name="c", subcore_axis_name="s")
    f = pl.kernel(
        _tec_body,
        mesh=mesh,
        out_type=jax.ShapeDtypeStruct((B,), jnp.float32),
        scratch_types=[
            pltpu.VMEM((BPW,), jnp.int32),
            pltpu.VMEM((BPW,), jnp.int32),
            pltpu.VMEM((BPW,), jnp.int32),
            pltpu.VMEM((CH, 2 * D), jnp.float32),
            pltpu.VMEM((CH, 2 * D), jnp.float32),
            pltpu.VMEM((CH, 2 * D), jnp.float32),
            pltpu.VMEM((BPW,), jnp.float32),
            pltpu.SemaphoreType.DMA,
        ],
    )
    return f(hid, lab, tid, ent2, rel2)


def kernel(head_ind, label, tail_ind, ent_embs, rel_embs):
    hid = head_ind.astype(jnp.int32)
    lab = label.astype(jnp.int32)
    tid = tail_ind.astype(jnp.int32)
    ent2 = ent_embs.reshape(ent_embs.shape[0] // 2, 2 * D)
    rel2 = rel_embs.reshape(rel_embs.shape[0] // 2, 2 * D)
    return _sc_transe(hid, lab, tid, ent2, rel2)


# speedup vs baseline: 12.4522x; 1.5917x over previous
import jax
import jax.numpy as jnp
from jax import lax
from jax.experimental import pallas as pl
from jax.experimental.pallas import tpu as pltpu
from jax.experimental.pallas import tpu_sc as plsc

B = 16384
D = 64
NC = 2
NS = 16
NW = NC * NS
BPW = B // NW
CH = 128
NCH = BPW // CH


def _rsqrt(x):
    i = lax.bitcast_convert_type(x, jnp.int32)
    i = jnp.int32(0x5F3759DF) - (i >> 1)
    y = lax.bitcast_convert_type(i, jnp.float32)
    for _ in range(3):
        y = y * (1.5 - 0.5 * x * y * y)
    return y


def _tec_body(hid_hbm, lab_hbm, tid_hbm, ent_hbm, rel_hbm, out_hbm,
              hidx, lidx, tidx, hrows, trows, rrows, outv, sem):
    wid = lax.axis_index("s") * NC + lax.axis_index("c")
    base = wid * BPW

    pltpu.sync_copy(hid_hbm.at[pl.ds(base, BPW)], hidx)
    pltpu.sync_copy(lab_hbm.at[pl.ds(base, BPW)], lidx)
    pltpu.sync_copy(tid_hbm.at[pl.ds(base, BPW)], tidx)

    lane = lax.broadcasted_iota(jnp.int32, (16,), 0)
    zero16 = jnp.zeros((16,), jnp.float32)
    bfly = [lane ^ k for k in (8, 4, 2, 1)]

    def hsum(x):
        for idx in bfly:
            x = x + x.at[idx].get(mode="promise_in_bounds", unique_indices=True)
        return x

    def fetch(ch, hrows, trows, rrows):
        cps = []
        for b in range(CH // 16):
            r0 = ch * CH + b * 16
            hv = hidx[pl.ds(r0, 16)]
            lv = lidx[pl.ds(r0, 16)]
            tv = tidx[pl.ds(r0, 16)]
            for j in range(16):
                row = b * 16 + j
                cps.append(pltpu.async_copy(ent_hbm.at[hv[j]], hrows.at[row], sem))
                cps.append(pltpu.async_copy(ent_hbm.at[tv[j]], trows.at[row], sem))
                cps.append(pltpu.async_copy(rel_hbm.at[lv[j]], rrows.at[row], sem))
        return cps

    def compute(ch, hrows, trows, rrows):
        for b in range(CH // 16):
            acc = [zero16] * 6
            for j in range(16):
                i = b * 16 + j
                h = [hrows[i, pl.ds(16 * c, 16)] for c in range(4)]
                t = [trows[i, pl.ds(16 * c, 16)] for c in range(4)]
                r = [rrows[i, pl.ds(16 * c, 16)] for c in range(4)]
                prods = [
                    sum(h[c] * h[c] for c in range(4)),
                    sum(t[c] * t[c] for c in range(4)),
                    sum(r[c] * r[c] for c in range(4)),
                    sum(h[c] * r[c] for c in range(4)),
                    sum(h[c] * t[c] for c in range(4)),
                    sum(t[c] * r[c] for c in range(4)),
                ]
                m = lane == j
                acc = [jnp.where(m, hsum(p), a) for p, a in zip(prods, acc)]
            hh, tt, rr, hr, ht, tr = acc
            ih = _rsqrt(jnp.maximum(hh, 1e-24))
            it = _rsqrt(jnp.maximum(tt, 1e-24))
            ssd = (hh * ih * ih + rr + tt * it * it
                   + 2.0 * (hr * ih) - 2.0 * (ht * (ih * it)) - 2.0 * (tr * it))
            ssd = jnp.maximum(ssd, 0.0)
            outv[pl.ds(ch * CH + b * 16, 16)] = ssd * _rsqrt(jnp.maximum(ssd, 1e-24))

    def chunk(ch, carry):
        cps = fetch(ch, hrows, trows, rrows)
        for cp in cps:
            cp.wait()
        compute(ch, hrows, trows, rrows)
        return carry

    lax.fori_loop(0, NCH, chunk, 0)

    pltpu.sync_copy(outv, out_hbm.at[pl.ds(base, BPW)])


@jax.jit
def _sc_transe(hid, lab, tid, ent_embs, rel_embs):
    mesh = plsc.VectorSubcoreMesh(core_axis_name="c", subcore_axis_name="s")
    f = pl.kernel(
        _tec_body,
        mesh=mesh,
        out_type=jax.ShapeDtypeStruct((B,), jnp.float32),
        scratch_types=[
            pltpu.VMEM((BPW,), jnp.int32),
            pltpu.VMEM((BPW,), jnp.int32),
            pltpu.VMEM((BPW,), jnp.int32),
            pltpu.VMEM((CH, D), jnp.float32),
            pltpu.VMEM((CH, D), jnp.float32),
            pltpu.VMEM((CH, D), jnp.float32),
            pltpu.VMEM((BPW,), jnp.float32),
            pltpu.SemaphoreType.DMA,
        ],
    )
    return f(hid, lab, tid, ent_embs, rel_embs)


def kernel(head_ind, label, tail_ind, ent_embs, rel_embs):
    hid = head_ind.astype(jnp.int32)
    lab = label.astype(jnp.int32)
    tid = tail_ind.astype(jnp.int32)
    return _sc_transe(hid, lab, tid, ent_embs, rel_embs)


# speedup vs baseline: 12.5260x; 1.0059x over previous
import jax
import jax.numpy as jnp
from jax import lax
from jax.experimental import pallas as pl
from jax.experimental.pallas import tpu as pltpu
from jax.experimental.pallas import tpu_sc as plsc

B = 16384
D = 64
NC = 2
NS = 16
NW = NC * NS
BPW = B // NW
CH = 128
NCH = BPW // CH


def _rsqrt(x):
    i = lax.bitcast_convert_type(x, jnp.int32)
    i = jnp.int32(0x5F3759DF) - (i >> 1)
    y = lax.bitcast_convert_type(i, jnp.float32)
    for _ in range(3):
        y = y * (1.5 - 0.5 * x * y * y)
    return y


def _tec_body(hid_hbm, lab_hbm, tid_hbm, ent_hbm, rel_hbm, out_hbm,
              hidx, lidx, tidx, hrows, trows, rrows, outv, *sems):
    wid = lax.axis_index("s") * NC + lax.axis_index("c")
    base = wid * BPW

    pltpu.sync_copy(hid_hbm.at[pl.ds(base, BPW)], hidx)
    pltpu.sync_copy(lab_hbm.at[pl.ds(base, BPW)], lidx)
    pltpu.sync_copy(tid_hbm.at[pl.ds(base, BPW)], tidx)

    lane = lax.broadcasted_iota(jnp.int32, (16,), 0)
    zero16 = jnp.zeros((16,), jnp.float32)
    bfly = [lane ^ k for k in (8, 4, 2, 1)]

    def hsum(x):
        for idx in bfly:
            x = x + x.at[idx].get(mode="promise_in_bounds", unique_indices=True)
        return x

    def fetch(ch, hrows, trows, rrows):
        cps = []
        for b in range(CH // 16):
            r0 = ch * CH + b * 16
            hv = hidx[pl.ds(r0, 16)]
            lv = lidx[pl.ds(r0, 16)]
            tv = tidx[pl.ds(r0, 16)]
            blk = []
            sem = sems[b]
            for j in range(16):
                row = b * 16 + j
                blk.append(pltpu.async_copy(ent_hbm.at[hv[j]], hrows.at[row], sem))
                blk.append(pltpu.async_copy(ent_hbm.at[tv[j]], trows.at[row], sem))
                blk.append(pltpu.async_copy(rel_hbm.at[lv[j]], rrows.at[row], sem))
            cps.append(blk)
        return cps

    def compute_block(ch, b, hrows, trows, rrows):
        if True:
            acc = [zero16] * 6
            for j in range(16):
                i = b * 16 + j
                h = [hrows[i, pl.ds(16 * c, 16)] for c in range(4)]
                t = [trows[i, pl.ds(16 * c, 16)] for c in range(4)]
                r = [rrows[i, pl.ds(16 * c, 16)] for c in range(4)]
                prods = [
                    sum(h[c] * h[c] for c in range(4)),
                    sum(t[c] * t[c] for c in range(4)),
                    sum(r[c] * r[c] for c in range(4)),
                    sum(h[c] * r[c] for c in range(4)),
                    sum(h[c] * t[c] for c in range(4)),
                    sum(t[c] * r[c] for c in range(4)),
                ]
                m = lane == j
                acc = [jnp.where(m, hsum(p), a) for p, a in zip(prods, acc)]
            hh, tt, rr, hr, ht, tr = acc
            ih = _rsqrt(jnp.maximum(hh, 1e-24))
            it = _rsqrt(jnp.maximum(tt, 1e-24))
            ssd = (hh * ih * ih + rr + tt * it * it
                   + 2.0 * (hr * ih) - 2.0 * (ht * (ih * it)) - 2.0 * (tr * it))
            ssd = jnp.maximum(ssd, 0.0)
            outv[pl.ds(ch * CH + b * 16, 16)] = ssd * _rsqrt(jnp.maximum(ssd, 1e-24))

    def chunk(ch, carry):
        cps = fetch(ch, hrows, trows, rrows)
        for b in range(CH // 16):
            for cp in cps[b]:
                cp.wait()
            compute_block(ch, b, hrows, trows, rrows)
        return carry

    lax.fori_loop(0, NCH, chunk, 0)

    pltpu.sync_copy(outv, out_hbm.at[pl.ds(base, BPW)])


@jax.jit
def _sc_transe(hid, lab, tid, ent_embs, rel_embs):
    mesh = plsc.VectorSubcoreMesh(core_axis_name="c", subcore_axis_name="s")
    f = pl.kernel(
        _tec_body,
        mesh=mesh,
        out_type=jax.ShapeDtypeStruct((B,), jnp.float32),
        scratch_types=[
            pltpu.VMEM((BPW,), jnp.int32),
            pltpu.VMEM((BPW,), jnp.int32),
            pltpu.VMEM((BPW,), jnp.int32),
            pltpu.VMEM((CH, D), jnp.float32),
            pltpu.VMEM((CH, D), jnp.float32),
            pltpu.VMEM((CH, D), jnp.float32),
            pltpu.VMEM((BPW,), jnp.float32),
        ] + [pltpu.SemaphoreType.DMA] * (CH // 16),
    )
    return f(hid, lab, tid, ent_embs, rel_embs)


def kernel(head_ind, label, tail_ind, ent_embs, rel_embs):
    hid = head_ind.astype(jnp.int32)
    lab = label.astype(jnp.int32)
    tid = tail_ind.astype(jnp.int32)
    return _sc_transe(hid, lab, tid, ent_embs, rel_embs)
